# Initial kernel scaffold; baseline (speedup 1.0000x reference)
import jax
import jax.numpy as jnp
from jax.experimental import pallas as pl


def kernel(cls_score, bbox_pred):
    raise NotImplementedError("write your pallas kernel here")


# speedup vs baseline: 24.7496x; 24.7496x over previous
import functools

import jax
import jax.numpy as jnp
import numpy as np
from jax.experimental import pallas as pl

_SCORE_THR = 0.05
_IOU_THR = 0.5
_MAX_NUM = 100
_STRIDE = 8.0
_EPS = 1e-6


def _nms_body(img_max, n_valid, cls_ref, bb_ref, px_ref, py_ref, out_ref):
    raw = jax.nn.sigmoid(cls_ref[0])
    dl = jnp.exp(bb_ref[0, 0]) * _STRIDE
    dt = jnp.exp(bb_ref[0, 1]) * _STRIDE
    dr = jnp.exp(bb_ref[0, 2]) * _STRIDE
    db = jnp.exp(bb_ref[0, 3]) * _STRIDE
    px = px_ref[...]
    py = py_ref[...]

    x1 = jnp.clip(px - dl, 0.0, img_max)
    y1 = jnp.clip(py - dt, 0.0, img_max)
    x2 = jnp.clip(px + dr, 0.0, img_max)
    y2 = jnp.clip(py + db, 0.0, img_max)

    cx = jnp.minimum(dl, dr) / (jnp.maximum(dl, dr) + _EPS)
    cy = jnp.minimum(dt, db) / (jnp.maximum(dt, db) + _EPS)
    ctr = jnp.sqrt(jnp.clip(cx * cy, 0.0, None))

    s = jnp.where(raw > _SCORE_THR, raw * ctr, 0.0)

    shape = raw.shape
    row_i = jax.lax.broadcasted_iota(jnp.int32, shape, 0)
    col_i = jax.lax.broadcasted_iota(jnp.int32, shape, 1)
    kf = (row_i * 128 + col_i).astype(jnp.float32)
    s = jnp.where(kf < n_valid, s, 0.0)

    area = jnp.clip(x2 - x1, 0.0, None) * jnp.clip(y2 - y1, 0.0, None)

    lane = jax.lax.broadcasted_iota(jnp.int32, (1, 128), 1)

    def body(i, carry):
        s, ax1, ay1, ax2, ay2, asc = carry
        m = jnp.max(s)
        idx = jnp.min(jnp.where(s == m, kf, 3.0e7))
        one = kf == idx
        bx1 = jnp.sum(jnp.where(one, x1, 0.0))
        by1 = jnp.sum(jnp.where(one, y1, 0.0))
        bx2 = jnp.sum(jnp.where(one, x2, 0.0))
        by2 = jnp.sum(jnp.where(one, y2, 0.0))
        ix1 = jnp.maximum(bx1, x1)
        iy1 = jnp.maximum(by1, y1)
        ix2 = jnp.minimum(bx2, x2)
        iy2 = jnp.minimum(by2, y2)
        inter = jnp.clip(ix2 - ix1, 0.0, None) * jnp.clip(iy2 - iy1, 0.0, None)
        ba = jnp.clip(bx2 - bx1, 0.0, None) * jnp.clip(by2 - by1, 0.0, None)
        iou = inter / (ba + area - inter + _EPS)
        sup = jnp.where(iou > _IOU_THR, 0.0, s)
        sup = jnp.where(one, 0.0, sup)
        valid = m > 0.0
        sel = lane == i
        ax1 = jnp.where(sel, jnp.where(valid, bx1, 0.0), ax1)
        ay1 = jnp.where(sel, jnp.where(valid, by1, 0.0), ay1)
        ax2 = jnp.where(sel, jnp.where(valid, bx2, 0.0), ax2)
        ay2 = jnp.where(sel, jnp.where(valid, by2, 0.0), ay2)
        asc = jnp.where(sel, jnp.where(valid, m, 0.0), asc)
        return (sup, ax1, ay1, ax2, ay2, asc)

    z = jnp.zeros((1, 128), jnp.float32)
    _, ax1, ay1, ax2, ay2, asc = jax.lax.fori_loop(
        0, _MAX_NUM, body, (s, z, z, z, z, z))

    out_ref[0, 0:1, :] = ax1
    out_ref[0, 1:2, :] = ay1
    out_ref[0, 2:3, :] = ax2
    out_ref[0, 3:4, :] = ay2
    out_ref[0, 4:5, :] = asc
    out_ref[0, 5:8, :] = jnp.zeros((3, 128), jnp.float32)


@jax.jit
def kernel(cls_score, bbox_pred):
    B, C, H, W = cls_score.shape
    N = H * W
    R = (N + 127) // 128
    R = ((R + 7) // 8) * 8
    NP = R * 128
    img_max = float(H) * _STRIDE

    cls_flat = cls_score.reshape(B, N)
    cls_flat = jnp.pad(cls_flat, ((0, 0), (0, NP - N)), constant_values=-30.0)
    cls_flat = cls_flat.reshape(B, R, 128)

    bb_flat = bbox_pred.reshape(B, 4, N)
    bb_flat = jnp.pad(bb_flat, ((0, 0), (0, 0), (0, NP - N)))
    bb_flat = bb_flat.reshape(B, 4, R, 128)

    k = np.arange(NP)
    ix = (k % W).astype(np.float32)
    iy = (k // W).astype(np.float32)
    px = jnp.asarray(((ix + 0.5) * _STRIDE).reshape(R, 128))
    py = jnp.asarray(((iy + 0.5) * _STRIDE).reshape(R, 128))

    out = pl.pallas_call(
        functools.partial(_nms_body, img_max, float(N)),
        grid=(B,),
        in_specs=[
            pl.BlockSpec((1, R, 128), lambda b: (b, 0, 0)),
            pl.BlockSpec((1, 4, R, 128), lambda b: (b, 0, 0, 0)),
            pl.BlockSpec((R, 128), lambda b: (0, 0)),
            pl.BlockSpec((R, 128), lambda b: (0, 0)),
        ],
        out_specs=pl.BlockSpec((1, 8, 128), lambda b: (b, 0, 0)),
        out_shape=jax.ShapeDtypeStruct((B, 8, 128), jnp.float32),
    )(cls_flat, bb_flat, px, py)

    det = out[:, :5, :_MAX_NUM].transpose(0, 2, 1)
    labels = jnp.zeros((B, _MAX_NUM), jnp.int32)
    return det, labels


# speedup vs baseline: 30.1732x; 1.2191x over previous
import jax
import jax.numpy as jnp
import numpy as np
from jax.experimental import pallas as pl
from jax.experimental.pallas import tpu as pltpu

_SCORE_THR = 0.05
_IOU_THR = 0.5
_MAX_NUM = 100
_STRIDE = 8.0
_EPS = 1e-6
_BIG = 3.0e7


def _argmin_idx(s, m, kf):
    return jnp.min(jnp.where(s == m, kf, _BIG))


def _nms_body(img_max, n_valid, B, cls_ref, bb_ref, px_ref, py_ref,
              out_ref, sx1, sy1, sx2, sy2, sar, skf, ss):
    shape = px_ref.shape
    row_i = jax.lax.broadcasted_iota(jnp.int32, shape, 0)
    col_i = jax.lax.broadcasted_iota(jnp.int32, shape, 1)
    kf = (row_i * 128 + col_i).astype(jnp.float32)
    skf[...] = kf
    px = px_ref[...]
    py = py_ref[...]

    for b in range(B):
        raw = jax.nn.sigmoid(cls_ref[b])
        dl = jnp.exp(bb_ref[b, 0]) * _STRIDE
        dt = jnp.exp(bb_ref[b, 1]) * _STRIDE
        dr = jnp.exp(bb_ref[b, 2]) * _STRIDE
        db = jnp.exp(bb_ref[b, 3]) * _STRIDE

        x1 = jnp.clip(px - dl, 0.0, img_max)
        y1 = jnp.clip(py - dt, 0.0, img_max)
        x2 = jnp.clip(px + dr, 0.0, img_max)
        y2 = jnp.clip(py + db, 0.0, img_max)

        cx = jnp.minimum(dl, dr) / (jnp.maximum(dl, dr) + _EPS)
        cy = jnp.minimum(dt, db) / (jnp.maximum(dt, db) + _EPS)
        ctr = jnp.sqrt(jnp.clip(cx * cy, 0.0, None))

        s = jnp.where(raw > _SCORE_THR, raw * ctr, 0.0)
        s = jnp.where(kf < n_valid, s, 0.0)

        sx1[b] = x1
        sy1[b] = y1
        sx2[b] = x2
        sy2[b] = y2
        sar[b] = jnp.clip(x2 - x1, 0.0, None) * jnp.clip(y2 - y1, 0.0, None)
        ss[b] = s

    lane_i = jax.lax.broadcasted_iota(jnp.int32, (1, 128), 1)

    ms = []
    idxs = []
    for b in range(B):
        s = ss[b]
        m = jnp.max(s)
        ms.append(m)
        idxs.append(_argmin_idx(s, m, skf[...]))

    zero = jnp.zeros((1, 128), jnp.float32)
    accs = [[zero] * 5 for _ in range(B)]

    def body(i, carry):
        ms, idxs, accs = carry
        new_ms = []
        new_idxs = []
        new_accs = []
        for b in range(B):
            m = ms[b]
            idx = idxs[b]
            ii = idx.astype(jnp.int32)
            row = ii >> 7
            lane = ii & 127
            onerow = lane_i == lane
            rx1 = jnp.sum(jnp.where(onerow, sx1[b, pl.ds(row, 1), :], 0.0))
            ry1 = jnp.sum(jnp.where(onerow, sy1[b, pl.ds(row, 1), :], 0.0))
            rx2 = jnp.sum(jnp.where(onerow, sx2[b, pl.ds(row, 1), :], 0.0))
            ry2 = jnp.sum(jnp.where(onerow, sy2[b, pl.ds(row, 1), :], 0.0))

            x1 = sx1[b]
            y1 = sy1[b]
            x2 = sx2[b]
            y2 = sy2[b]
            area = sar[b]
            s = ss[b]
            kf = skf[...]

            ix1 = jnp.maximum(rx1, x1)
            iy1 = jnp.maximum(ry1, y1)
            ix2 = jnp.minimum(rx2, x2)
            iy2 = jnp.minimum(ry2, y2)
            inter = (jnp.clip(ix2 - ix1, 0.0, None)
                     * jnp.clip(iy2 - iy1, 0.0, None))
            ba = (jnp.clip(rx2 - rx1, 0.0, None)
                  * jnp.clip(ry2 - ry1, 0.0, None))
            iou = inter / (ba + area - inter + _EPS)
            kill = (iou > _IOU_THR) | (kf == idx)
            sup = jnp.where(kill, 0.0, s)
            ss[b] = sup

            nm = jnp.max(sup)
            new_ms.append(nm)
            new_idxs.append(_argmin_idx(sup, nm, kf))

            valid = m > 0.0
            sel = lane_i == i
            vals = (rx1, ry1, rx2, ry2, m)
            new_accs.append([
                jnp.where(sel, jnp.where(valid, v, 0.0), a)
                for v, a in zip(vals, accs[b])])
        return (new_ms, new_idxs, new_accs)

    _, _, accs = jax.lax.fori_loop(0, _MAX_NUM, body, (ms, idxs, accs))

    for b in range(B):
        for c in range(5):
            out_ref[b, c:c + 1, :] = accs[b][c]
        out_ref[b, 5:8, :] = jnp.zeros((3, 128), jnp.float32)


@jax.jit
def kernel(cls_score, bbox_pred):
    B, C, H, W = cls_score.shape
    N = H * W
    R = (N + 127) // 128
    R = ((R + 7) // 8) * 8
    NP = R * 128
    img_max = float(H) * _STRIDE

    cls_flat = cls_score.reshape(B, N)
    cls_flat = jnp.pad(cls_flat, ((0, 0), (0, NP - N)), constant_values=-30.0)
    cls_flat = cls_flat.reshape(B, R, 128)

    bb_flat = bbox_pred.reshape(B, 4, N)
    bb_flat = jnp.pad(bb_flat, ((0, 0), (0, 0), (0, NP - N)))
    bb_flat = bb_flat.reshape(B, 4, R, 128)

    k = np.arange(NP)
    ix = (k % W).astype(np.float32)
    iy = (k // W).astype(np.float32)
    px = jnp.asarray(((ix + 0.5) * _STRIDE).reshape(R, 128))
    py = jnp.asarray(((iy + 0.5) * _STRIDE).reshape(R, 128))

    def body(*refs):
        _nms_body(img_max, float(N), B, *refs)

    scratch = [pltpu.VMEM((B, R, 128), jnp.float32)] * 5
    scratch.append(pltpu.VMEM((R, 128), jnp.float32))
    scratch.append(pltpu.VMEM((B, R, 128), jnp.float32))
    out = pl.pallas_call(
        body,
        out_specs=pl.BlockSpec((B, 8, 128), lambda: (0, 0, 0)),
        out_shape=jax.ShapeDtypeStruct((B, 8, 128), jnp.float32),
        in_specs=[
            pl.BlockSpec((B, R, 128), lambda: (0, 0, 0)),
            pl.BlockSpec((B, 4, R, 128), lambda: (0, 0, 0, 0)),
            pl.BlockSpec((R, 128), lambda: (0, 0)),
            pl.BlockSpec((R, 128), lambda: (0, 0)),
        ],
        scratch_shapes=tuple(scratch),
    )(cls_flat, bb_flat, px, py)

    det = out[:, :5, :_MAX_NUM].transpose(0, 2, 1)
    labels = jnp.zeros((B, _MAX_NUM), jnp.int32)
    return det, labels
